# baseline (speedup 1.0000x reference)
import jax
import jax.numpy as jnp
from jax import lax
from jax.experimental import pallas as pl
from jax.experimental.pallas import tpu as pltpu
from jax.experimental.pallas import tpu_sc as plsc

NCLS = 80
NCHAN = 85
GAMMA = 2.0
NTGT = 4096
CH = 128
CPS = NTGT // CH
NCHUNK = 3 * CPS
NC, NS = 2, 16
NTILE = NC * NS
WSCALE = (128, 64, 32)
HWS = tuple(w * w for w in WSCALE)
CELLS = tuple(16 * hw for hw in HWS)
CELL_OFF = (0, CELLS[0], CELLS[0] + CELLS[1])
NCELL = sum(CELLS)
ZSPAN = 2048
ZPER = 11
MAPW = NS * ZPER * ZSPAN
SLAB = NCHAN * CH


def _sc_body_a(p0, idxh, cellh, out0, outm,
               idxA, cellT, gbufA, mbuf, zbuf, dbuf, ones_v, cntmap,
               semg, semz, semm):
    cid = lax.axis_index("c")
    sid = lax.axis_index("s")
    wid = sid * NC + cid

    @pl.loop(0, ZSPAN // 16)
    def _(i):
        zbuf[pl.ds(i * 16, 16)] = jnp.zeros((16,), jnp.float32)

    @pl.loop(0, CH // 16)
    def _(i):
        ones_v[pl.ds(i * 16, 16)] = jnp.ones((16,), jnp.float32)

    pltpu.sync_copy(idxh.at[wid], idxA)

    @pl.loop(0, NCHAN)
    def _(c):
        pltpu.async_copy(p0.at[idxA.at[c]], gbufA.at[c], semg)

    @pl.loop(0, ZPER)
    def _(i):
        pltpu.async_copy(zbuf,
                         cntmap.at[pl.ds((sid * ZPER + i) * ZSPAN, ZSPAN)],
                         semz)
    pltpu.make_async_copy(p0.at[pl.ds(0, ZPER * ZSPAN)], dbuf, semz).wait()
    plsc.subcore_barrier()

    @pl.loop(0, NCHUNK // NS)
    def _(k):
        jj = sid * (NCHUNK // NS) + k
        pltpu.sync_copy(cellh.at[jj], cellT)
        pltpu.sync_copy(ones_v, cntmap.at[cellT], add=True)

    plsc.subcore_barrier()

    pltpu.make_async_copy(p0.at[pl.ds(0, SLAB)],
                          dbuf.at[pl.ds(0, SLAB)], semg).wait()
    pltpu.sync_copy(cellh.at[wid], cellT)
    pltpu.async_copy(cntmap.at[cellT], gbufA.at[NCHAN], semm).wait()
    pltpu.sync_copy(gbufA, out0.at[wid])

    for s_const in (1, 2):
        jj = (s_const - 1) * CPS + wid
        pltpu.sync_copy(cellh.at[CPS * s_const + wid], cellT)
        pltpu.async_copy(cntmap.at[cellT], mbuf, semm).wait()
        pltpu.sync_copy(mbuf, outm.at[jj])


def _sc_body_b(t1, t2, rowh, out12, rbufA, rbufB, slabA, slabB, dbuf, semh):
    cid = lax.axis_index("c")
    sid = lax.axis_index("s")
    wid = sid * NC + cid

    pltpu.sync_copy(rowh.at[wid], rbufA)
    pltpu.sync_copy(rowh.at[CPS + wid], rbufB)
    da = pltpu.async_copy(t1.at[rbufA], slabA, semh)
    db = pltpu.async_copy(t2.at[rbufB], slabB, semh)
    da.wait()
    db.wait()
    pltpu.sync_copy(slabA, out12.at[wid])
    pltpu.sync_copy(slabB, out12.at[CPS + wid])


def _tc_body(pl0, t1v, t2v, g0, g12, gm, par, obox, ocls, oobj, otot):
    i = pl.program_id(0)

    def bce0(x):
        return jnp.maximum(x, 0.0) + jnp.log(1.0 + jnp.exp(-jnp.abs(x)))

    @pl.when(i == 0)
    def _():
        f32 = jnp.float32
        p = lambda k: par[k, 0:CPS, :]
        x0, x1, x2, x3, x4 = (g0[:, c, :] for c in range(5))
        mult = g0[:, NCHAN, :]
        px = (1.0 / (1.0 + jnp.exp(-x0)) + p(6)) / p(8)
        py = (1.0 / (1.0 + jnp.exp(-x1)) + p(7)) / p(9)
        pw = jnp.exp(jnp.minimum(x2, 4.0)) / p(8)
        ph = jnp.exp(jnp.minimum(x3, 4.0)) / p(9)
        l1 = (jnp.abs(px - p(0)) + jnp.abs(py - p(1))
              + jnp.abs(pw - p(2)) + jnp.abs(ph - p(3))) * 0.25
        valid = p(10)
        box_sum = jnp.sum(jnp.where(valid > 0, l1 * p(4), 0.0))
        xc = g0[:, 5:5 + NCLS, :]
        lane_c = lax.broadcasted_iota(jnp.int32, (CPS, NCLS, CH), 1).astype(f32)
        onehot = (lane_c == p(5)[:, None, :]).astype(f32)
        bcec = (jnp.maximum(xc, 0.0) - xc * onehot
                + jnp.log(1.0 + jnp.exp(-jnp.abs(xc))))
        cls_sum = jnp.sum(jnp.where(valid[:, None, :] > 0, bcec, 0.0)) / NCLS
        corr = jnp.sum(jnp.where(valid > 0, x4 / mult * p(11), 0.0))

        q = lambda k: par[k, CPS:NCHUNK, :]
        y0 = g12[:, :, 0]
        y1 = g12[:, :, 1]
        y2 = g12[:, :, 2]
        y3 = g12[:, :, 3]
        y4 = g12[:, :, 4]
        qx = (1.0 / (1.0 + jnp.exp(-y0)) + q(6)) / q(8)
        qy = (1.0 / (1.0 + jnp.exp(-y1)) + q(7)) / q(9)
        qw = jnp.exp(jnp.minimum(y2, 4.0)) / q(8)
        qh = jnp.exp(jnp.minimum(y3, 4.0)) / q(9)
        l1q = (jnp.abs(qx - q(0)) + jnp.abs(qy - q(1))
               + jnp.abs(qw - q(2)) + jnp.abs(qh - q(3))) * 0.25
        validq = q(10)
        box_sum += jnp.sum(jnp.where(validq > 0, l1q * q(4), 0.0))
        yc = g12[:, :, 5:5 + NCLS]
        lane_q = lax.broadcasted_iota(
            jnp.int32, (2 * CPS, CH, NCLS), 2).astype(f32)
        onehot_q = (lane_q == q(5)[:, :, None]).astype(f32)
        bceq = (jnp.maximum(yc, 0.0) - yc * onehot_q
                + jnp.log(1.0 + jnp.exp(-jnp.abs(yc))))
        cls_sum += jnp.sum(jnp.where(validq[:, :, None] > 0, bceq, 0.0)) / NCLS
        corr += jnp.sum(jnp.where(validq > 0, y4 / gm[...] * q(11), 0.0))

        obox[0, 0] = box_sum / NTGT
        ocls[0, 0] = cls_sum / NTGT
        oobj[0, 0] = -corr

    oobj[0, 0] += (jnp.sum(bce0(pl0[0, 0])) / (16.0 * HWS[0])
                   + jnp.sum(bce0(t1v[:, 4:5])) / (16.0 * HWS[1])
                   + jnp.sum(bce0(t2v[:, 4:5])) / (16.0 * HWS[2]))

    @pl.when(i == 15)
    def _():
        otot[0, 0] = obox[0, 0] + ocls[0, 0] + oobj[0, 0]


def kernel(preds_0, preds_1, preds_2, targets):
    f32 = jnp.float32
    t = lax.stop_gradient(targets)
    b = t[:, 0].astype(jnp.int32)
    clsf = t[:, 1]
    cx, cy, bw, bh = t[:, 2], t[:, 3], t[:, 4], t[:, 5]
    area = jnp.maximum(bw * bh, 1e-6)
    sidx = jnp.where(area <= 0.01, 0, jnp.where(area <= 0.03, 1, 2)).astype(jnp.int32)
    weight = 1.0 + GAMMA * (1.0 - jnp.sqrt(area))

    wsa = jnp.array(WSCALE, jnp.int32).reshape(3, 1)
    hwa = jnp.array(HWS, jnp.int32).reshape(3, 1)
    offa = jnp.array(CELL_OFF, jnp.int32).reshape(3, 1)
    wsf3 = wsa.astype(f32)
    gx3 = jnp.clip((cx[None] * wsf3).astype(jnp.int32), 0, wsa - 1)
    gy3 = jnp.clip((cy[None] * wsf3).astype(jnp.int32), 0, wsa - 1)
    lcell3 = (b[None] * wsa + gy3) * wsa + gx3
    cell3 = offa + lcell3
    mask3 = sidx[None] == jnp.arange(3, dtype=jnp.int32)[:, None]

    gid = jnp.arange(3 * NTGT, dtype=jnp.int32).reshape(3, NTGT)
    cell3 = jnp.where(mask3, cell3, NCELL + gid % (MAPW - NCELL - 8))
    cells_arr = cell3.reshape(NCHUNK, CH)

    base0 = (b * NCHAN * 128 + gy3[0]) * 128 + gx3[0]
    idx0 = (base0.reshape(CPS, 1, CH)
            + jnp.arange(NCHAN, dtype=jnp.int32).reshape(1, NCHAN, 1) * HWS[0])
    rows = lcell3[1:].reshape(2 * CPS, CH)

    maskf = mask3.astype(f32)

    def brd(v):
        return jnp.broadcast_to(v[None], (3, NTGT))

    par = jnp.stack([
        brd(cx), brd(cy), brd(bw), brd(bh), brd(weight), brd(clsf),
        gx3.astype(f32), gy3.astype(f32),
        jnp.broadcast_to(wsf3, (3, NTGT)), jnp.broadcast_to(wsf3, (3, NTGT)),
        maskf,
        maskf / (16.0 * hwa.astype(f32)),
    ]).reshape(12, NCHUNK, CH)

    t1v = preds_1.transpose(0, 2, 3, 1).reshape(16 * HWS[1], NCHAN)
    t2v = preds_2.transpose(0, 2, 3, 1).reshape(16 * HWS[2], NCHAN)
    t1 = jnp.pad(t1v, ((0, 0), (0, 128 - NCHAN)))
    t2 = jnp.pad(t2v, ((0, 0), (0, 128 - NCHAN)))

    sca = pl.kernel(
        _sc_body_a,
        out_type=[
            jax.ShapeDtypeStruct((CPS, 88, CH), f32),
            jax.ShapeDtypeStruct((2 * CPS, CH), f32),
        ],
        mesh=plsc.VectorSubcoreMesh(core_axis_name="c", subcore_axis_name="s"),
        scratch_types=[
            pltpu.VMEM((NCHAN, CH), jnp.int32),
            pltpu.VMEM((CH,), jnp.int32),
            pltpu.VMEM((88, CH), f32),
            pltpu.VMEM((CH,), f32),
            pltpu.VMEM((ZSPAN,), f32),
            pltpu.VMEM((ZPER * ZSPAN,), f32),
            pltpu.VMEM((CH,), f32),
            pltpu.VMEM_SHARED((MAPW,), f32),
            pltpu.SemaphoreType.DMA,
            pltpu.SemaphoreType.DMA,
            pltpu.SemaphoreType.DMA,
        ],
    )
    scb = pl.kernel(
        _sc_body_b,
        out_type=jax.ShapeDtypeStruct((2 * CPS, CH, 128), f32),
        mesh=plsc.VectorSubcoreMesh(core_axis_name="c", subcore_axis_name="s"),
        scratch_types=[
            pltpu.VMEM((CH,), jnp.int32),
            pltpu.VMEM((CH,), jnp.int32),
            pltpu.VMEM((CH, 128), f32),
            pltpu.VMEM((CH, 128), f32),
            pltpu.VMEM((ZPER * ZSPAN,), f32),
            pltpu.SemaphoreType.DMA,
        ],
    )
    g0, gm = sca(preds_0.reshape(-1), idx0, cells_arr)
    g12 = scb(t1, t2, rows)

    losses = pl.pallas_call(
        _tc_body,
        grid=(16,),
        in_specs=[
            pl.BlockSpec((1, 1, 128, 128), lambda i: (i, 4, 0, 0)),
            pl.BlockSpec((16 * HWS[1] // 16, NCHAN), lambda i: (i, 0)),
            pl.BlockSpec((16 * HWS[2] // 16, NCHAN), lambda i: (i, 0)),
            pl.BlockSpec((CPS, 88, CH), lambda i: (0, 0, 0)),
            pl.BlockSpec((2 * CPS, CH, 128), lambda i: (0, 0, 0)),
            pl.BlockSpec((2 * CPS, CH), lambda i: (0, 0)),
            pl.BlockSpec((12, NCHUNK, CH), lambda i: (0, 0, 0)),
        ],
        out_specs=[pl.BlockSpec((1, 1), lambda i: (0, 0),
                                memory_space=pltpu.SMEM)] * 4,
        out_shape=[jax.ShapeDtypeStruct((1, 1), f32)] * 4,
    )(preds_0, t1v, t2v, g0, g12, gm, par)
    obox, ocls, oobj, otot = losses
    return otot[0, 0], obox[0, 0], oobj[0, 0], ocls[0, 0]
